# Initial kernel scaffold; baseline (speedup 1.0000x reference)
import jax
import jax.numpy as jnp
from jax.experimental import pallas as pl


def kernel(pos, cell):
    raise NotImplementedError("write your pallas kernel here")


# speedup vs baseline: 7.9693x; 7.9693x over previous
import functools

import jax
import jax.numpy as jnp
from jax.experimental import pallas as pl

_N = 4096
_K = 17
_KPAD = 32
_R = 256


def _knn_block(frac_q_ref, frac_kT_ref, cell_ref, idx_ref, d2_ref):
    i = pl.program_id(0)
    q = frac_q_ref[...]
    kT = frac_kT_ref[...]
    col = jax.lax.broadcasted_iota(jnp.int32, (_R, _N), 1)
    row = i * _R + jax.lax.broadcasted_iota(jnp.int32, (_R, _N), 0)

    acc = jnp.zeros((_R, _N), dtype=jnp.float32)
    for d in range(3):
        diff = q[:, d:d + 1] - kT[d:d + 1, :]
        diff = diff - jnp.where(diff > 0.5, 1.0, 0.0) + jnp.where(diff < -0.5, 1.0, 0.0)
        w = diff * cell_ref[0, d]
        acc = acc + w * w
    acc = jnp.where(col == row, acc + 1e10, acc)

    idx_cols = []
    d2_cols = []
    for _ in range(_K):
        m = jnp.min(acc, axis=1, keepdims=True)
        sel = jnp.min(jnp.where(acc == m, col, _N), axis=1, keepdims=True)
        idx_cols.append(sel)
        d2_cols.append(m)
        acc = jnp.where(col == sel, jnp.float32(3e10), acc)
    pad_i = jnp.zeros((_R, _KPAD - _K), dtype=jnp.int32)
    pad_f = jnp.zeros((_R, _KPAD - _K), dtype=jnp.float32)
    idx_ref[...] = jnp.concatenate(idx_cols + [pad_i], axis=1)
    d2_ref[...] = jnp.concatenate(d2_cols + [pad_f], axis=1)


def kernel(pos, cell):
    n = pos.shape[0]
    frac = pos / cell
    frac_q = jnp.pad(frac, ((0, 0), (0, 5)))
    frac_kT = jnp.pad(frac.T, ((0, 5), (0, 0)))
    cell_pad = jnp.pad(cell, (0, 125)).reshape(1, 128)

    grid = (n // _R,)
    idx, d2 = pl.pallas_call(
        _knn_block,
        grid=grid,
        in_specs=[
            pl.BlockSpec((_R, 8), lambda i: (i, 0)),
            pl.BlockSpec((8, _N), lambda i: (0, 0)),
            pl.BlockSpec((1, 128), lambda i: (0, 0)),
        ],
        out_specs=[
            pl.BlockSpec((_R, _KPAD), lambda i: (i, 0)),
            pl.BlockSpec((_R, _KPAD), lambda i: (i, 0)),
        ],
        out_shape=[
            jax.ShapeDtypeStruct((n, _KPAD), jnp.int32),
            jax.ShapeDtypeStruct((n, _KPAD), jnp.float32),
        ],
    )(frac_q, frac_kT, cell_pad)

    idx = idx[:, :_K]
    d2 = d2[:, :_K]
    dist = jnp.sqrt(jnp.maximum(d2, 0.0) + 1e-12)
    src = idx.reshape(-1)
    dst = jnp.repeat(jnp.arange(n), _K)
    edge_index = jnp.stack([src, dst]).astype(jnp.int32)
    return edge_index, dist.reshape(-1)
